# Initial kernel scaffold; baseline (speedup 1.0000x reference)
import jax
import jax.numpy as jnp
from jax.experimental import pallas as pl


def kernel(pred_logits, pred_boxes, orig_target_sizes):
    raise NotImplementedError("write your pallas kernel here")


# speedup vs baseline: 5.7655x; 5.7655x over previous
import functools

import jax
import jax.numpy as jnp
from jax.experimental import pallas as pl
from jax.experimental.pallas import tpu as pltpu

_A = 20000
_C = 80
_K = 300
_LANES = 128
_ROWS = (_A * _C) // _LANES
_GROUP = 128
_NGROUP = -(-_ROWS // _GROUP)
_ROWS_PAD = _NGROUP * _GROUP
_M2_PAD = -(-_NGROUP // 8) * 8
_OUTW = 512
_NEG = float("-inf")
_BIG = 2**30


def _topk_kernel(logits_ref, boxes_ref, vals_ref, idxs_ref, outb_ref,
                 data_ref, m2_ref):
    data_ref[pl.ds(0, _ROWS), :] = logits_ref[0]
    data_ref[pl.ds(_ROWS, _ROWS_PAD - _ROWS), :] = jnp.full(
        (_ROWS_PAD - _ROWS, _LANES), _NEG, jnp.float32)

    m2 = jnp.max(data_ref[...].reshape(_NGROUP, _GROUP, _LANES), axis=1)
    m2_ref[pl.ds(0, _NGROUP), :] = m2
    m2_ref[pl.ds(_NGROUP, _M2_PAD - _NGROUP), :] = jnp.full(
        (_M2_PAD - _NGROUP, _LANES), _NEG, jnp.float32)

    g_iota = jax.lax.broadcasted_iota(jnp.int32, (_M2_PAD, _LANES), 0)
    rl_blk = jax.lax.broadcasted_iota(jnp.int32, (_GROUP, _LANES), 0) * _LANES \
        + jax.lax.broadcasted_iota(jnp.int32, (_GROUP, _LANES), 1)
    lane_row = jax.lax.broadcasted_iota(jnp.int32, (1, _LANES), 1)
    out_iota = jax.lax.broadcasted_iota(jnp.int32, (1, _OUTW), 1)

    def body(k, carry):
        vals, idxs = carry
        m2_all = m2_ref[...]
        m = jnp.max(m2_all)
        g = jnp.min(jnp.where(m2_all >= m, g_iota, _BIG))
        base = g * _GROUP
        block = data_ref[pl.ds(base, _GROUP), :]
        rl = jnp.min(jnp.where(block >= m, rl_blk, _BIG))
        r = rl // _LANES
        lane = rl - r * _LANES
        flat = base * _LANES + rl

        rowv = data_ref[pl.ds(base + r, 1), :]
        data_ref[pl.ds(base + r, 1), :] = jnp.where(
            lane_row == lane, _NEG, rowv)
        m2_ref[pl.ds(g, 1), :] = jnp.max(
            data_ref[pl.ds(base, _GROUP), :], axis=0, keepdims=True)

        vals = jnp.where(out_iota == k, m, vals)
        idxs = jnp.where(out_iota == k, flat, idxs)

        q = flat // _C
        outb_ref[0, pl.ds(k, 1), :] = boxes_ref[0, pl.ds(q, 1), :]
        return vals, idxs

    init = (jnp.full((1, _OUTW), _NEG, jnp.float32),
            jnp.zeros((1, _OUTW), jnp.int32))
    vals, idxs = jax.lax.fori_loop(0, _K, body, init)
    vals_ref[0] = vals
    idxs_ref[0] = idxs


@jax.jit
def kernel(pred_logits, pred_boxes, orig_target_sizes):
    n = pred_logits.shape[0]
    logits = pred_logits.reshape(n, _ROWS, _LANES)

    vals, idxs, rawb = pl.pallas_call(
        _topk_kernel,
        grid=(n,),
        in_specs=[
            pl.BlockSpec((1, _ROWS, _LANES), lambda i: (i, 0, 0)),
            pl.BlockSpec((1, _A, 4), lambda i: (i, 0, 0)),
        ],
        out_specs=[
            pl.BlockSpec((1, 1, _OUTW), lambda i: (i, 0, 0)),
            pl.BlockSpec((1, 1, _OUTW), lambda i: (i, 0, 0)),
            pl.BlockSpec((1, _OUTW, 4), lambda i: (i, 0, 0)),
        ],
        out_shape=[
            jax.ShapeDtypeStruct((n, 1, _OUTW), jnp.float32),
            jax.ShapeDtypeStruct((n, 1, _OUTW), jnp.int32),
            jax.ShapeDtypeStruct((n, _OUTW, 4), jnp.float32),
        ],
        scratch_shapes=[
            pltpu.VMEM((_ROWS_PAD, _LANES), jnp.float32),
            pltpu.VMEM((_M2_PAD, _LANES), jnp.float32),
        ],
    )(logits, pred_boxes)

    scores = jax.nn.sigmoid(vals[:, 0, :_K])
    index = idxs[:, 0, :_K, None]
    labels = index - (index // _C) * _C

    raw = rawb[:, :_K, :]
    cx, cy, w, h = raw[..., 0], raw[..., 1], raw[..., 2], raw[..., 3]
    x1 = cx - 0.5 * w
    y1 = cy - 0.5 * h
    x2 = cx + 0.5 * w
    y2 = cy + 0.5 * h
    xyxy = jnp.stack([x1, y1, x2, y2], axis=-1)
    scale = jnp.tile(orig_target_sizes, (1, 2))[:, None, :]
    boxes = xyxy * scale
    return labels, boxes, scores


# speedup vs baseline: 5.7659x; 1.0001x over previous
import functools

import jax
import jax.numpy as jnp
from jax.experimental import pallas as pl
from jax.experimental.pallas import tpu as pltpu

_A = 20000
_C = 80
_K = 300
_LANES = 128
_ROWS = (_A * _C) // _LANES
_GROUP = 128
_NGROUP = -(-_ROWS // _GROUP)
_ROWS_PAD = _NGROUP * _GROUP
_M2_PAD = -(-_NGROUP // 8) * 8
_OUTW = 512
_NEG = float("-inf")
_BIG = 2**30


def _topk_kernel(logits_ref, boxes_ref, vals_ref, idxs_ref, outb_ref,
                 data_ref, m2_ref):
    data_ref[pl.ds(0, _ROWS), :] = logits_ref[0]
    data_ref[pl.ds(_ROWS, _ROWS_PAD - _ROWS), :] = jnp.full(
        (_ROWS_PAD - _ROWS, _LANES), _NEG, jnp.float32)

    m2 = jnp.max(data_ref[...].reshape(_NGROUP, _GROUP, _LANES), axis=1)
    m2_ref[pl.ds(0, _NGROUP), :] = m2
    m2_ref[pl.ds(_NGROUP, _M2_PAD - _NGROUP), :] = jnp.full(
        (_M2_PAD - _NGROUP, _LANES), _NEG, jnp.float32)

    g_iota = jax.lax.broadcasted_iota(jnp.int32, (_M2_PAD, _LANES), 0)
    rl_blk = jax.lax.broadcasted_iota(jnp.int32, (_GROUP, _LANES), 0) * _LANES \
        + jax.lax.broadcasted_iota(jnp.int32, (_GROUP, _LANES), 1)
    lane_row = jax.lax.broadcasted_iota(jnp.int32, (1, _LANES), 1)
    out_iota = jax.lax.broadcasted_iota(jnp.int32, (1, _OUTW), 1)

    def body(k, carry):
        vals, idxs = carry
        m2_all = m2_ref[...]
        m = jnp.max(m2_all)
        g = jnp.min(jnp.where(m2_all >= m, g_iota, _BIG))
        base = g * _GROUP
        block = data_ref[pl.ds(base, _GROUP), :]
        rl = jnp.min(jnp.where(block >= m, rl_blk, _BIG))
        r = rl // _LANES
        lane = rl - r * _LANES
        flat = base * _LANES + rl

        rowv = data_ref[pl.ds(base + r, 1), :]
        data_ref[pl.ds(base + r, 1), :] = jnp.where(
            lane_row == lane, _NEG, rowv)
        m2_ref[pl.ds(g, 1), :] = jnp.max(
            data_ref[pl.ds(base, _GROUP), :], axis=0, keepdims=True)

        vals = jnp.where(out_iota == k, m, vals)
        idxs = jnp.where(out_iota == k, flat, idxs)

        q = flat // _C
        outb_ref[0, pl.ds(k, 1), :] = boxes_ref[0, pl.ds(q, 1), :]
        return vals, idxs

    init = (jnp.full((1, _OUTW), _NEG, jnp.float32),
            jnp.zeros((1, _OUTW), jnp.int32))
    vals, idxs = jax.lax.fori_loop(0, _K, body, init)
    vals_ref[0] = vals
    idxs_ref[0] = idxs


@jax.jit
def kernel(pred_logits, pred_boxes, orig_target_sizes):
    n = pred_logits.shape[0]
    logits = pred_logits.reshape(n, _ROWS, _LANES)

    vals, idxs, rawb = pl.pallas_call(
        _topk_kernel,
        grid=(n,),
        in_specs=[
            pl.BlockSpec((1, _ROWS, _LANES), lambda i: (i, 0, 0)),
            pl.BlockSpec((1, _A, 4), lambda i: (i, 0, 0)),
        ],
        out_specs=[
            pl.BlockSpec((1, 1, _OUTW), lambda i: (i, 0, 0)),
            pl.BlockSpec((1, 1, _OUTW), lambda i: (i, 0, 0)),
            pl.BlockSpec((1, _OUTW, 4), lambda i: (i, 0, 0)),
        ],
        out_shape=[
            jax.ShapeDtypeStruct((n, 1, _OUTW), jnp.float32),
            jax.ShapeDtypeStruct((n, 1, _OUTW), jnp.int32),
            jax.ShapeDtypeStruct((n, _OUTW, 4), jnp.float32),
        ],
        scratch_shapes=[
            pltpu.VMEM((_ROWS_PAD, _LANES), jnp.float32),
            pltpu.VMEM((_M2_PAD, _LANES), jnp.float32),
        ],
        compiler_params=pltpu.CompilerParams(
            dimension_semantics=("parallel",)),
    )(logits, pred_boxes)

    scores = jax.nn.sigmoid(vals[:, 0, :_K])
    index = idxs[:, 0, :_K, None]
    labels = index - (index // _C) * _C

    raw = rawb[:, :_K, :]
    cx, cy, w, h = raw[..., 0], raw[..., 1], raw[..., 2], raw[..., 3]
    x1 = cx - 0.5 * w
    y1 = cy - 0.5 * h
    x2 = cx + 0.5 * w
    y2 = cy + 0.5 * h
    xyxy = jnp.stack([x1, y1, x2, y2], axis=-1)
    scale = jnp.tile(orig_target_sizes, (1, 2))[:, None, :]
    boxes = xyxy * scale
    return labels, boxes, scores


# speedup vs baseline: 5.8357x; 1.0121x over previous
import functools

import jax
import jax.numpy as jnp
from jax.experimental import pallas as pl
from jax.experimental.pallas import tpu as pltpu

_A = 20000
_C = 80
_K = 300
_LANES = 128
_ROWS = (_A * _C) // _LANES
_GROUP = 128
_NGROUP = -(-_ROWS // _GROUP)
_ROWS_PAD = _NGROUP * _GROUP
_M2_PAD = -(-_NGROUP // 8) * 8
_OUTW = 512
_NEG = float("-inf")
_BIG = 2**30


def _topk_kernel(logits_ref, boxes_ref, vals_ref, idxs_ref, outb_ref,
                 data_ref, m2_ref):
    data_ref[pl.ds(0, _ROWS), :] = logits_ref[0]
    data_ref[pl.ds(_ROWS, _ROWS_PAD - _ROWS), :] = jnp.full(
        (_ROWS_PAD - _ROWS, _LANES), _NEG, jnp.float32)

    m2 = jnp.max(data_ref[...].reshape(_NGROUP, _GROUP, _LANES), axis=1)
    m2_ref[pl.ds(0, _NGROUP), :] = m2
    m2_ref[pl.ds(_NGROUP, _M2_PAD - _NGROUP), :] = jnp.full(
        (_M2_PAD - _NGROUP, _LANES), _NEG, jnp.float32)

    g_iota = jax.lax.broadcasted_iota(jnp.int32, (_M2_PAD, _LANES), 0)
    rl_blk = jax.lax.broadcasted_iota(jnp.int32, (_GROUP, _LANES), 0) * _LANES \
        + jax.lax.broadcasted_iota(jnp.int32, (_GROUP, _LANES), 1)
    lane_row = jax.lax.broadcasted_iota(jnp.int32, (1, _LANES), 1)
    out_iota = jax.lax.broadcasted_iota(jnp.int32, (1, _OUTW), 1)

    def body(k, carry):
        vals, idxs = carry
        m2_all = m2_ref[...]
        m = jnp.max(m2_all)
        g = jnp.min(jnp.where(m2_all >= m, g_iota, _BIG))
        base = g * _GROUP
        block = data_ref[pl.ds(base, _GROUP), :]
        rl = jnp.min(jnp.where(block >= m, rl_blk, _BIG))
        r = rl // _LANES
        lane = rl - r * _LANES
        flat = base * _LANES + rl

        cleared = jnp.where(rl_blk == rl, _NEG, block)
        m2_ref[pl.ds(g, 1), :] = jnp.max(cleared, axis=0, keepdims=True)
        data_ref[pl.ds(base, _GROUP), :] = cleared

        vals = jnp.where(out_iota == k, m, vals)
        idxs = jnp.where(out_iota == k, flat, idxs)

        q = flat // _C
        outb_ref[0, pl.ds(k, 1), :] = boxes_ref[0, pl.ds(q, 1), :]
        return vals, idxs

    init = (jnp.full((1, _OUTW), _NEG, jnp.float32),
            jnp.zeros((1, _OUTW), jnp.int32))
    vals, idxs = jax.lax.fori_loop(0, _K, body, init)
    vals_ref[0] = vals
    idxs_ref[0] = idxs


@jax.jit
def kernel(pred_logits, pred_boxes, orig_target_sizes):
    n = pred_logits.shape[0]
    logits = pred_logits.reshape(n, _ROWS, _LANES)

    vals, idxs, rawb = pl.pallas_call(
        _topk_kernel,
        grid=(n,),
        in_specs=[
            pl.BlockSpec((1, _ROWS, _LANES), lambda i: (i, 0, 0)),
            pl.BlockSpec((1, _A, 4), lambda i: (i, 0, 0)),
        ],
        out_specs=[
            pl.BlockSpec((1, 1, _OUTW), lambda i: (i, 0, 0)),
            pl.BlockSpec((1, 1, _OUTW), lambda i: (i, 0, 0)),
            pl.BlockSpec((1, _OUTW, 4), lambda i: (i, 0, 0)),
        ],
        out_shape=[
            jax.ShapeDtypeStruct((n, 1, _OUTW), jnp.float32),
            jax.ShapeDtypeStruct((n, 1, _OUTW), jnp.int32),
            jax.ShapeDtypeStruct((n, _OUTW, 4), jnp.float32),
        ],
        scratch_shapes=[
            pltpu.VMEM((_ROWS_PAD, _LANES), jnp.float32),
            pltpu.VMEM((_M2_PAD, _LANES), jnp.float32),
        ],
        compiler_params=pltpu.CompilerParams(
            dimension_semantics=("parallel",)),
    )(logits, pred_boxes)

    scores = jax.nn.sigmoid(vals[:, 0, :_K])
    index = idxs[:, 0, :_K, None]
    labels = index - (index // _C) * _C

    raw = rawb[:, :_K, :]
    cx, cy, w, h = raw[..., 0], raw[..., 1], raw[..., 2], raw[..., 3]
    x1 = cx - 0.5 * w
    y1 = cy - 0.5 * h
    x2 = cx + 0.5 * w
    y2 = cy + 0.5 * h
    xyxy = jnp.stack([x1, y1, x2, y2], axis=-1)
    scale = jnp.tile(orig_target_sizes, (1, 2))[:, None, :]
    boxes = xyxy * scale
    return labels, boxes, scores
